# Initial kernel scaffold; baseline (speedup 1.0000x reference)
import jax
import jax.numpy as jnp
from jax.experimental import pallas as pl


def kernel(input_ids, bbox, token_type_ids, word_emb, pos_emb, tt_emb, x_emb, y_emb, h_emb, w_emb, Wx, bx, Wy, by, Wh, bh, Ww, bw, gamma, beta):
    raise NotImplementedError("write your pallas kernel here")


# speedup vs baseline: 3.4701x; 3.4701x over previous
import jax
import jax.numpy as jnp
from jax import lax
from jax.experimental import pallas as pl
from jax.experimental.pallas import tpu as pltpu
from jax.experimental.pallas import tpu_sc as plsc

B, S, H, HL = 4, 2048, 768, 128
N = B * S
EPS = 1e-12

NC, NS = 2, 16
NW = NC * NS
RW = N // NW
C = 64
NCHUNK = RW // C

BLK = 512


def _sc_gather_body(ids_hbm, b0_hbm, b1_hbm, b2_hbm, b3_hbm,
                    word_hbm, x_hbm, y_hbm, h_hbm, w_hbm,
                    words_out, xl_out, xr_out, yu_out, yl_out, he_out, we_out,
                    idw_v, i0_v, i1_v, i2_v, i3_v, d31_v, d20_v,
                    words_v, xl_v, xr_v, yu_v, yl_v, he_v, we_v, sem):
    wid = lax.axis_index("s") * NC + lax.axis_index("c")
    for c in range(NCHUNK):
        base = wid * RW + c * C
        sl = pl.ds(base, C)
        pltpu.sync_copy(ids_hbm.at[sl], idw_v)
        pltpu.sync_copy(b0_hbm.at[sl], i0_v)
        pltpu.sync_copy(b1_hbm.at[sl], i1_v)
        pltpu.sync_copy(b2_hbm.at[sl], i2_v)
        pltpu.sync_copy(b3_hbm.at[sl], i3_v)
        for k in range(C // 16):
            v = pl.ds(k * 16, 16)
            d31_v[v] = i3_v[v] - i1_v[v]
            d20_v[v] = i2_v[v] - i0_v[v]
        cps = [
            pltpu.async_copy(word_hbm.at[idw_v], words_v, sem),
            pltpu.async_copy(x_hbm.at[i0_v], xl_v, sem),
            pltpu.async_copy(x_hbm.at[i2_v], xr_v, sem),
            pltpu.async_copy(y_hbm.at[i1_v], yu_v, sem),
            pltpu.async_copy(y_hbm.at[i3_v], yl_v, sem),
            pltpu.async_copy(h_hbm.at[d31_v], he_v, sem),
            pltpu.async_copy(w_hbm.at[d20_v], we_v, sem),
        ]
        for cp in cps:
            cp.wait()
        pltpu.sync_copy(words_v, words_out.at[sl])
        pltpu.sync_copy(xl_v, xl_out.at[sl])
        pltpu.sync_copy(xr_v, xr_out.at[sl])
        pltpu.sync_copy(yu_v, yu_out.at[sl])
        pltpu.sync_copy(yl_v, yl_out.at[sl])
        pltpu.sync_copy(he_v, he_out.at[sl])
        pltpu.sync_copy(we_v, we_out.at[sl])


def _sc_gather(ids, b0, b1, b2, b3, word_emb, x_emb, y_emb, h_emb, w_emb):
    f32 = jnp.float32
    out_type = (
        jax.ShapeDtypeStruct((N, H), f32),
        jax.ShapeDtypeStruct((N, HL), f32),
        jax.ShapeDtypeStruct((N, HL), f32),
        jax.ShapeDtypeStruct((N, HL), f32),
        jax.ShapeDtypeStruct((N, HL), f32),
        jax.ShapeDtypeStruct((N, HL), f32),
        jax.ShapeDtypeStruct((N, HL), f32),
    )
    scratch = (
        [pltpu.VMEM((C,), jnp.int32) for _ in range(7)]
        + [pltpu.VMEM((C, H), f32)]
        + [pltpu.VMEM((C, HL), f32) for _ in range(6)]
        + [pltpu.SemaphoreType.DMA]
    )
    k = pl.kernel(
        _sc_gather_body,
        out_type=out_type,
        mesh=plsc.VectorSubcoreMesh(
            core_axis_name="c", subcore_axis_name="s",
            num_cores=NC, num_subcores=NS),
        scratch_types=scratch,
    )
    return k(ids, b0, b1, b2, b3, word_emb, x_emb, y_emb, h_emb, w_emb)


def _tc_body(words_ref, xl_ref, xr_ref, yu_ref, yl_ref, he_ref, we_ref,
             pos_ref, ttoh_ref, vecs_ref, wx_ref, wy_ref, wh_ref, ww_ref,
             out_ref):
    f32 = jnp.float32
    xs = xl_ref[...] + xr_ref[...]
    ys = yu_ref[...] + yl_ref[...]
    acc = jnp.dot(xs, wx_ref[...], preferred_element_type=f32)
    acc = acc + jnp.dot(ys, wy_ref[...], preferred_element_type=f32)
    acc = acc + jnp.dot(he_ref[...], wh_ref[...], preferred_element_type=f32)
    acc = acc + jnp.dot(we_ref[...], ww_ref[...], preferred_element_type=f32)
    vecs = vecs_ref[...]
    bias = 2.0 * (vecs[0:1, :] + vecs[1:2, :]) + vecs[2:3, :] + vecs[3:4, :]
    acc = acc + jnp.dot(ttoh_ref[...], vecs, preferred_element_type=f32)
    acc = acc + bias + words_ref[...] + pos_ref[...]
    mu = jnp.mean(acc, axis=1, keepdims=True)
    xc = acc - mu
    var = jnp.mean(xc * xc, axis=1, keepdims=True)
    out_ref[...] = xc * lax.rsqrt(var + EPS) * vecs[4:5, :] + vecs[5:6, :]


def _tc_combine(words, xl, xr, yu, yl, he, we, pos_emb, ttoh, vecs,
                Wx, Wy, Wh, Ww):
    grid = (N // BLK,)
    wide = pl.BlockSpec((BLK, H), lambda i: (i, 0))
    narrow = pl.BlockSpec((BLK, HL), lambda i: (i, 0))
    return pl.pallas_call(
        _tc_body,
        grid=grid,
        in_specs=[
            wide, narrow, narrow, narrow, narrow, narrow, narrow,
            pl.BlockSpec((BLK, H), lambda i: (i % (S // BLK), 0)),
            pl.BlockSpec((BLK, 8), lambda i: (i, 0)),
            pl.BlockSpec((8, H), lambda i: (0, 0)),
            pl.BlockSpec((HL, H), lambda i: (0, 0)),
            pl.BlockSpec((HL, H), lambda i: (0, 0)),
            pl.BlockSpec((HL, H), lambda i: (0, 0)),
            pl.BlockSpec((HL, H), lambda i: (0, 0)),
        ],
        out_specs=wide,
        out_shape=jax.ShapeDtypeStruct((N, H), jnp.float32),
    )(words, xl, xr, yu, yl, he, we, pos_emb, ttoh, vecs, Wx, Wy, Wh, Ww)


def kernel(input_ids, bbox, token_type_ids, word_emb, pos_emb, tt_emb,
           x_emb, y_emb, h_emb, w_emb, Wx, bx, Wy, by, Wh, bh, Ww, bw,
           gamma, beta):
    ids = input_ids.reshape(N)
    b0 = bbox[:, :, 0].reshape(N)
    b1 = bbox[:, :, 1].reshape(N)
    b2 = bbox[:, :, 2].reshape(N)
    b3 = bbox[:, :, 3].reshape(N)
    words, xl, xr, yu, yl, he, we = _sc_gather(
        ids, b0, b1, b2, b3, word_emb, x_emb, y_emb, h_emb, w_emb)
    ttoh = jax.nn.one_hot(token_type_ids.reshape(N) + 6, 8, dtype=jnp.float32)
    vecs = jnp.stack([bx, by, bh, bw, gamma, beta, tt_emb[0], tt_emb[1]])
    out = _tc_combine(words, xl, xr, yu, yl, he, we, pos_emb, ttoh, vecs,
                      Wx, Wy, Wh, Ww)
    return out.reshape(B, S, H)


# speedup vs baseline: 3.8459x; 1.1083x over previous
import jax
import jax.numpy as jnp
from jax import lax
from jax.experimental import pallas as pl
from jax.experimental.pallas import tpu as pltpu
from jax.experimental.pallas import tpu_sc as plsc

B, S, H, HL = 4, 2048, 768, 128
N = B * S
EPS = 1e-12

NC, NS = 2, 16
NW = NC * NS
RW = N // NW
C = 32
NCHUNK = RW // C

BLK = 512


def _sc_gather_body(ids_hbm, b0_hbm, b1_hbm, b2_hbm, b3_hbm,
                    word_hbm, x_hbm, y_hbm, h_hbm, w_hbm,
                    words_out, xl_out, xr_out, yu_out, yl_out, he_out, we_out,
                    idw_v, i0_v, i1_v, i2_v, i3_v, d31_v, d20_v,
                    words_v, xl_v, xr_v, yu_v, yl_v, he_v, we_v,
                    gsem0, gsem1, wsem0, wsem1):
    wid = lax.axis_index("s") * NC + lax.axis_index("c")
    base0 = wid * RW
    full = pl.ds(base0, RW)
    pltpu.sync_copy(ids_hbm.at[full], idw_v)
    pltpu.sync_copy(b0_hbm.at[full], i0_v)
    pltpu.sync_copy(b1_hbm.at[full], i1_v)
    pltpu.sync_copy(b2_hbm.at[full], i2_v)
    pltpu.sync_copy(b3_hbm.at[full], i3_v)
    for k in range(RW // 16):
        v = pl.ds(k * 16, 16)
        d31_v[v] = i3_v[v] - i1_v[v]
        d20_v[v] = i2_v[v] - i0_v[v]

    gsem = (gsem0, gsem1)
    wsem = (wsem0, wsem1)

    def issue_gathers(c, b):
        i = pl.ds(c * C, C)
        return [
            pltpu.async_copy(word_hbm.at[idw_v.at[i]], words_v.at[b], gsem[b]),
            pltpu.async_copy(x_hbm.at[i0_v.at[i]], xl_v.at[b], gsem[b]),
            pltpu.async_copy(x_hbm.at[i2_v.at[i]], xr_v.at[b], gsem[b]),
            pltpu.async_copy(y_hbm.at[i1_v.at[i]], yu_v.at[b], gsem[b]),
            pltpu.async_copy(y_hbm.at[i3_v.at[i]], yl_v.at[b], gsem[b]),
            pltpu.async_copy(h_hbm.at[d31_v.at[i]], he_v.at[b], gsem[b]),
            pltpu.async_copy(w_hbm.at[d20_v.at[i]], we_v.at[b], gsem[b]),
        ]

    def issue_writes(c, b):
        o = pl.ds(base0 + c * C, C)
        return [
            pltpu.async_copy(words_v.at[b], words_out.at[o], wsem[b]),
            pltpu.async_copy(xl_v.at[b], xl_out.at[o], wsem[b]),
            pltpu.async_copy(xr_v.at[b], xr_out.at[o], wsem[b]),
            pltpu.async_copy(yu_v.at[b], yu_out.at[o], wsem[b]),
            pltpu.async_copy(yl_v.at[b], yl_out.at[o], wsem[b]),
            pltpu.async_copy(he_v.at[b], he_out.at[o], wsem[b]),
            pltpu.async_copy(we_v.at[b], we_out.at[o], wsem[b]),
        ]

    gh = issue_gathers(0, 0)
    wh = [None, None]
    for c in range(NCHUNK):
        b = c & 1
        nb = b ^ 1
        ghn = None
        if c + 1 < NCHUNK:
            if wh[nb] is not None:
                for h in wh[nb]:
                    h.wait()
                wh[nb] = None
            ghn = issue_gathers(c + 1, nb)
        for h in gh:
            h.wait()
        wh[b] = issue_writes(c, b)
        gh = ghn
    for hs in wh:
        if hs is not None:
            for h in hs:
                h.wait()


def _sc_gather(ids, b0, b1, b2, b3, word_emb, x_emb, y_emb, h_emb, w_emb):
    f32 = jnp.float32
    out_type = (
        jax.ShapeDtypeStruct((N, H), f32),
        jax.ShapeDtypeStruct((N, HL), f32),
        jax.ShapeDtypeStruct((N, HL), f32),
        jax.ShapeDtypeStruct((N, HL), f32),
        jax.ShapeDtypeStruct((N, HL), f32),
        jax.ShapeDtypeStruct((N, HL), f32),
        jax.ShapeDtypeStruct((N, HL), f32),
    )
    scratch = (
        [pltpu.VMEM((RW,), jnp.int32) for _ in range(7)]
        + [pltpu.VMEM((2, C, H), f32)]
        + [pltpu.VMEM((2, C, HL), f32) for _ in range(6)]
        + [pltpu.SemaphoreType.DMA for _ in range(4)]
    )
    k = pl.kernel(
        _sc_gather_body,
        out_type=out_type,
        mesh=plsc.VectorSubcoreMesh(
            core_axis_name="c", subcore_axis_name="s",
            num_cores=NC, num_subcores=NS),
        scratch_types=scratch,
    )
    return k(ids, b0, b1, b2, b3, word_emb, x_emb, y_emb, h_emb, w_emb)


def _tc_body(words_ref, xl_ref, xr_ref, yu_ref, yl_ref, he_ref, we_ref,
             pos_ref, ttoh_ref, vecs_ref, wx_ref, wy_ref, wh_ref, ww_ref,
             out_ref):
    f32 = jnp.float32
    xs = xl_ref[...] + xr_ref[...]
    ys = yu_ref[...] + yl_ref[...]
    acc = jnp.dot(xs, wx_ref[...], preferred_element_type=f32)
    acc = acc + jnp.dot(ys, wy_ref[...], preferred_element_type=f32)
    acc = acc + jnp.dot(he_ref[...], wh_ref[...], preferred_element_type=f32)
    acc = acc + jnp.dot(we_ref[...], ww_ref[...], preferred_element_type=f32)
    vecs = vecs_ref[...]
    bias = 2.0 * (vecs[0:1, :] + vecs[1:2, :]) + vecs[2:3, :] + vecs[3:4, :]
    acc = acc + jnp.dot(ttoh_ref[...], vecs, preferred_element_type=f32)
    acc = acc + bias + words_ref[...] + pos_ref[...]
    mu = jnp.mean(acc, axis=1, keepdims=True)
    xc = acc - mu
    var = jnp.mean(xc * xc, axis=1, keepdims=True)
    out_ref[...] = xc * lax.rsqrt(var + EPS) * vecs[4:5, :] + vecs[5:6, :]


def _tc_combine(words, xl, xr, yu, yl, he, we, pos_emb, ttoh, vecs,
                Wx, Wy, Wh, Ww):
    grid = (S // BLK, B)
    row = lambda i, j: (j * (S // BLK) + i, 0)
    wide = pl.BlockSpec((BLK, H), row)
    narrow = pl.BlockSpec((BLK, HL), row)
    return pl.pallas_call(
        _tc_body,
        grid=grid,
        in_specs=[
            wide, narrow, narrow, narrow, narrow, narrow, narrow,
            pl.BlockSpec((BLK, H), lambda i, j: (i, 0)),
            pl.BlockSpec((BLK, 8), row),
            pl.BlockSpec((8, H), lambda i, j: (0, 0)),
            pl.BlockSpec((HL, H), lambda i, j: (0, 0)),
            pl.BlockSpec((HL, H), lambda i, j: (0, 0)),
            pl.BlockSpec((HL, H), lambda i, j: (0, 0)),
            pl.BlockSpec((HL, H), lambda i, j: (0, 0)),
        ],
        out_specs=wide,
        out_shape=jax.ShapeDtypeStruct((N, H), jnp.float32),
    )(words, xl, xr, yu, yl, he, we, pos_emb, ttoh, vecs, Wx, Wy, Wh, Ww)


def kernel(input_ids, bbox, token_type_ids, word_emb, pos_emb, tt_emb,
           x_emb, y_emb, h_emb, w_emb, Wx, bx, Wy, by, Wh, bh, Ww, bw,
           gamma, beta):
    ids = input_ids.reshape(N)
    b0 = bbox[:, :, 0].reshape(N)
    b1 = bbox[:, :, 1].reshape(N)
    b2 = bbox[:, :, 2].reshape(N)
    b3 = bbox[:, :, 3].reshape(N)
    words, xl, xr, yu, yl, he, we = _sc_gather(
        ids, b0, b1, b2, b3, word_emb, x_emb, y_emb, h_emb, w_emb)
    ttoh = jax.nn.one_hot(token_type_ids.reshape(N) + 6, 8, dtype=jnp.float32)
    vecs = jnp.stack([bx, by, bh, bw, gamma, beta, tt_emb[0], tt_emb[1]])
    out = _tc_combine(words, xl, xr, yu, yl, he, we, pos_emb, ttoh, vecs,
                      Wx, Wy, Wh, Ww)
    return out.reshape(B, S, H)
